# baseline (speedup 1.0000x reference)
import functools

import jax
import jax.numpy as jnp
from jax import lax
from jax.experimental import pallas as pl
from jax.experimental.pallas import tpu as pltpu
from jax.experimental.pallas import tpu_sc as plsc

_NC = 2
_NS = 16
_NW = _NC * _NS

_TR = 32
_CW = _TR * 1024


def _sc_body(B, L, D, x_hbm, pe_hbm, o_hbm, pe_buf, x_buf):
    w = lax.axis_index("s") * _NC + lax.axis_index("c")
    rows_per_w = L // _NW
    pe_off = w * rows_per_w * D
    n_chunks = rows_per_w * D // _CW
    for c in range(n_chunks):
        coff = pe_off + c * _CW
        pltpu.sync_copy(pe_hbm.at[pl.ds(coff, _CW)], pe_buf)
        for b in range(B):
            xoff = b * L * D + coff
            pltpu.sync_copy(x_hbm.at[pl.ds(xoff, _CW)], x_buf)

            def vec_add(i, _):
                base = i * 128
                for u in range(8):
                    s = pl.ds(base + u * 16, 16)
                    x_buf[s] = x_buf[s] + pe_buf[s]
                return 0

            lax.fori_loop(0, _CW // 128, vec_add, 0)
            pltpu.sync_copy(x_buf, o_hbm.at[pl.ds(xoff, _CW)])


def kernel(x, pe):
    B, L, D = x.shape
    xf = x.reshape(B * L * D)
    pef = pe.reshape(pe.shape[0] * D)
    mesh = plsc.VectorSubcoreMesh(core_axis_name="c", subcore_axis_name="s")
    sc_call = functools.partial(
        pl.kernel,
        mesh=mesh,
        out_type=jax.ShapeDtypeStruct((B * L * D,), x.dtype),
        scratch_types=[
            pltpu.VMEM((_CW,), jnp.float32),
            pltpu.VMEM((_CW,), jnp.float32),
        ],
    )(functools.partial(_sc_body, B, L, D))
    out = sc_call(xf, pef)
    return out.reshape(B, L, D)


# speedup vs baseline: 1.1539x; 1.1539x over previous
import functools

import jax
import jax.numpy as jnp
from jax import lax
from jax.experimental import pallas as pl
from jax.experimental.pallas import tpu as pltpu
from jax.experimental.pallas import tpu_sc as plsc

_NC = 2
_NS = 16
_NW = _NC * _NS

_TR = 32
_CW = _TR * 1024


def _sc_body(B, L, D, x_hbm, pe_hbm, o_hbm, pe_buf, xb0, xb1,
             si0, si1, so0, so1):
    w = lax.axis_index("s") * _NC + lax.axis_index("c")
    rows_per_w = L // _NW
    pe_off = w * rows_per_w * D
    n_chunks = rows_per_w * D // _CW
    nsteps = n_chunks * B
    xbufs = (xb0, xb1)
    sin = (si0, si1)
    sout = (so0, so1)

    def xoff(t):
        c, b = divmod(t, B)
        return b * L * D + pe_off + c * _CW

    pltpu.async_copy(x_hbm.at[pl.ds(xoff(0), _CW)], xbufs[0], sin[0])
    for t in range(nsteps):
        c, b = divmod(t, B)
        cur = t % 2
        if b == 0:
            pltpu.sync_copy(pe_hbm.at[pl.ds(pe_off + c * _CW, _CW)], pe_buf)
        pltpu.make_async_copy(
            x_hbm.at[pl.ds(xoff(t), _CW)], xbufs[cur], sin[cur]).wait()
        if t + 1 < nsteps:
            if t >= 1:
                pltpu.make_async_copy(
                    xbufs[1 - cur], o_hbm.at[pl.ds(xoff(t - 1), _CW)],
                    sout[1 - cur]).wait()
            pltpu.async_copy(
                x_hbm.at[pl.ds(xoff(t + 1), _CW)], xbufs[1 - cur],
                sin[1 - cur])

        xb = xbufs[cur]

        @plsc.parallel_loop(0, _CW, 16, unroll=8)
        def _vec_add(i):
            s = pl.ds(i, 16)
            xb[s] = xb[s] + pe_buf[s]

        pltpu.async_copy(xb, o_hbm.at[pl.ds(xoff(t), _CW)], sout[cur])

    for t in (nsteps - 2, nsteps - 1):
        pltpu.make_async_copy(
            xbufs[t % 2], o_hbm.at[pl.ds(xoff(t), _CW)], sout[t % 2]).wait()


def kernel(x, pe):
    B, L, D = x.shape
    xf = x.reshape(B * L * D)
    pef = pe.reshape(pe.shape[0] * D)
    mesh = plsc.VectorSubcoreMesh(core_axis_name="c", subcore_axis_name="s")
    sc_call = functools.partial(
        pl.kernel,
        mesh=mesh,
        out_type=jax.ShapeDtypeStruct((B * L * D,), x.dtype),
        scratch_types=[
            pltpu.VMEM((_CW,), jnp.float32),
            pltpu.VMEM((_CW,), jnp.float32),
            pltpu.VMEM((_CW,), jnp.float32),
            pltpu.SemaphoreType.DMA,
            pltpu.SemaphoreType.DMA,
            pltpu.SemaphoreType.DMA,
            pltpu.SemaphoreType.DMA,
        ],
    )(functools.partial(_sc_body, B, L, D))
    out = sc_call(xf, pef)
    return out.reshape(B, L, D)
